# baseline (speedup 1.0000x reference)
import functools

import numpy as np
import jax
import jax.numpy as jnp
from jax.experimental import pallas as pl
from jax.experimental.pallas import tpu as pltpu

_N = 4096
_D = 2048
_F = 4096
_E = 8
_MC = 5
_DROP_P = 0.1

_TN = 1024
_TF = 512
_CN = 256


def _rotl32(x, d):
    return ((x << np.uint32(d)) | (x >> np.uint32(32 - d))).astype(np.uint32)


def _threefry2x32(k1, k2, x0, x1):
    k1 = np.uint32(k1)
    k2 = np.uint32(k2)
    ks = [k1, k2, np.uint32(k1 ^ k2 ^ np.uint32(0x1BD11BDA))]
    x0 = (x0 + ks[0]).astype(np.uint32)
    x1 = (x1 + ks[1]).astype(np.uint32)
    rots = [(13, 15, 26, 6), (17, 29, 16, 24)]
    krot = [ks[1], ks[2], ks[0]]
    for i in range(5):
        for d in rots[0]:
            x0 = (x0 + x1).astype(np.uint32)
            x1 = _rotl32(x1, d)
            x1 = (x1 ^ x0).astype(np.uint32)
        x0 = (x0 + krot[0]).astype(np.uint32)
        x1 = (x1 + krot[1] + np.uint32(i + 1)).astype(np.uint32)
        krot = krot[1:] + krot[:1]
        rots = rots[1:] + rots[:1]
    return x0, x1


@functools.lru_cache(maxsize=None)
def _dropout_masks():
    root = np.array([0, 42], dtype=np.uint32)
    n = _N * _F
    lo = np.arange(n, dtype=np.uint32)
    hi = np.zeros(n, dtype=np.uint32)
    out = np.empty((_MC, _N, _F), dtype=np.int8)
    for i in range(_MC):
        a, b = _threefry2x32(root[0], root[1],
                             np.array([0], np.uint32),
                             np.array([i], np.uint32))
        k1, k2 = a[0], b[0]
        b1_, b2_ = _threefry2x32(k1, k2, hi, lo)
        bits = b1_ ^ b2_
        u = ((bits >> np.uint32(9)) | np.uint32(0x3F800000)).view(np.float32)
        keep = (u - np.float32(1.0)) < np.float32(1.0 - _DROP_P)
        out[i] = keep.reshape(_N, _F).astype(np.int8)
    return out


def _body(z_ref, w1_ref, b1_ref, w2_ref, b2_ref, m_ref, ent_ref, idx_ref,
          acc_ref):
    f = pl.program_id(1)
    nf = pl.num_programs(1)

    w2 = w2_ref[...] * (1.0 / (1.0 - _DROP_P))
    b1row = b1_ref[...]

    for c in range(_TN // _CN):
        sl = slice(c * _CN, (c + 1) * _CN)
        hc = jnp.dot(z_ref[sl, :], w1_ref[...],
                     preferred_element_type=jnp.float32)
        hc = jnp.maximum(hc + b1row, 0.0)
        for i in range(_MC):
            g = hc * m_ref[i, sl, :].astype(jnp.float32)
            p_i = jnp.dot(g, w2, preferred_element_type=jnp.float32)

            @pl.when(f == 0)
            def _(p_i=p_i, i=i, sl=sl):
                acc_ref[i, sl, :] = p_i

            @pl.when(f != 0)
            def _(p_i=p_i, i=i, sl=sl):
                acc_ref[i, sl, :] += p_i

    @pl.when(f == nf - 1)
    def _():
        preds = acc_ref[...] + b2_ref[...]
        mean = jnp.mean(preds, axis=0)
        dev = preds - mean[None]
        var = jnp.sum(dev * dev, axis=0) * (1.0 / (_MC - 1))
        ent_ref[...] = var
        mn = jnp.min(var, axis=-1, keepdims=True)
        eid = jax.lax.broadcasted_iota(jnp.int32, (_TN, _E), 1)
        idx = jnp.min(jnp.where(var == mn, eid, _E), axis=-1)
        idx_ref[...] = idx.reshape(_TN, 1)


def kernel(z, W1, b1, W2, b2):
    masks = _dropout_masks()
    b1r = b1.reshape(1, _F)
    b2r = b2.reshape(1, _E)

    grid = (_N // _TN, _F // _TF)
    ent, idx = pl.pallas_call(
        _body,
        grid=grid,
        in_specs=[
            pl.BlockSpec((_TN, _D), lambda n, f: (n, 0)),
            pl.BlockSpec((_D, _TF), lambda n, f: (0, f)),
            pl.BlockSpec((1, _TF), lambda n, f: (0, f)),
            pl.BlockSpec((_TF, _E), lambda n, f: (f, 0)),
            pl.BlockSpec((1, _E), lambda n, f: (0, 0)),
            pl.BlockSpec((_MC, _TN, _TF), lambda n, f: (0, n, f)),
        ],
        out_specs=[
            pl.BlockSpec((_TN, _E), lambda n, f: (n, 0)),
            pl.BlockSpec((_TN, 1), lambda n, f: (n, 0)),
        ],
        out_shape=[
            jax.ShapeDtypeStruct((_N, _E), jnp.float32),
            jax.ShapeDtypeStruct((_N, 1), jnp.int32),
        ],
        scratch_shapes=[pltpu.VMEM((_MC, _TN, _E), jnp.float32)],
        compiler_params=pltpu.CompilerParams(
            dimension_semantics=("parallel", "arbitrary"),
        ),
    )(z, W1, b1r, W2, b2r, masks)
    return idx.reshape(_N), ent


# speedup vs baseline: 1.4278x; 1.4278x over previous
import functools

import numpy as np
import jax
import jax.numpy as jnp
from jax.experimental import pallas as pl
from jax.experimental.pallas import tpu as pltpu

_N = 4096
_D = 2048
_F = 4096
_E = 8
_MC = 5
_DROP_P = 0.1

_TN = 1024
_TF = 512
_CN = 256


def _rotl32(x, d):
    return ((x << np.uint32(d)) | (x >> np.uint32(32 - d))).astype(np.uint32)


def _threefry2x32(k1, k2, x0, x1):
    k1 = np.uint32(k1)
    k2 = np.uint32(k2)
    ks = [k1, k2, np.uint32(k1 ^ k2 ^ np.uint32(0x1BD11BDA))]
    x0 = (x0 + ks[0]).astype(np.uint32)
    x1 = (x1 + ks[1]).astype(np.uint32)
    rots = [(13, 15, 26, 6), (17, 29, 16, 24)]
    krot = [ks[1], ks[2], ks[0]]
    for i in range(5):
        for d in rots[0]:
            x0 = (x0 + x1).astype(np.uint32)
            x1 = _rotl32(x1, d)
            x1 = (x1 ^ x0).astype(np.uint32)
        x0 = (x0 + krot[0]).astype(np.uint32)
        x1 = (x1 + krot[1] + np.uint32(i + 1)).astype(np.uint32)
        krot = krot[1:] + krot[:1]
        rots = rots[1:] + rots[:1]
    return x0, x1


@functools.lru_cache(maxsize=None)
def _dropout_masks():
    root = np.array([0, 42], dtype=np.uint32)
    n = _N * _F
    lo = np.arange(n, dtype=np.uint32)
    hi = np.zeros(n, dtype=np.uint32)
    out = np.empty((_MC, _N, _F), dtype=np.int8)
    for i in range(_MC):
        a, b = _threefry2x32(root[0], root[1],
                             np.array([0], np.uint32),
                             np.array([i], np.uint32))
        k1, k2 = a[0], b[0]
        b1_, b2_ = _threefry2x32(k1, k2, hi, lo)
        bits = b1_ ^ b2_
        u = ((bits >> np.uint32(9)) | np.uint32(0x3F800000)).view(np.float32)
        keep = (u - np.float32(1.0)) < np.float32(1.0 - _DROP_P)
        out[i] = keep.reshape(_N, _F).astype(np.int8)
    return out


def _body(z_ref, w1_ref, b1_ref, w2_ref, b2_ref, m_ref, ent_ref, idx_ref,
          acc_ref):
    f = pl.program_id(1)
    nf = pl.num_programs(1)

    h = jnp.dot(z_ref[...], w1_ref[...], preferred_element_type=jnp.float32)
    h = jnp.maximum(h + b1_ref[...], 0.0) / (1.0 - _DROP_P)

    w2 = w2_ref[...]
    for i in range(_MC):
        g = h * m_ref[i].astype(jnp.float32)
        p_i = jnp.dot(g, w2, preferred_element_type=jnp.float32)

        @pl.when(f == 0)
        def _(p_i=p_i, i=i):
            acc_ref[i] = p_i

        @pl.when(f != 0)
        def _(p_i=p_i, i=i):
            acc_ref[i] += p_i

    @pl.when(f == nf - 1)
    def _():
        preds = acc_ref[...] + b2_ref[...]
        mean = jnp.mean(preds, axis=0)
        dev = preds - mean[None]
        var = jnp.sum(dev * dev, axis=0) * (1.0 / (_MC - 1))
        ent_ref[...] = var
        mn = jnp.min(var, axis=-1, keepdims=True)
        eid = jax.lax.broadcasted_iota(jnp.int32, (_TN, _E), 1)
        idx = jnp.min(jnp.where(var == mn, eid, _E), axis=-1)
        idx_ref[...] = idx.reshape(_TN, 1)


def kernel(z, W1, b1, W2, b2):
    masks = _dropout_masks()
    b1r = b1.reshape(1, _F)
    b2r = b2.reshape(1, _E)

    grid = (_N // _TN, _F // _TF)
    ent, idx = pl.pallas_call(
        _body,
        grid=grid,
        in_specs=[
            pl.BlockSpec((_TN, _D), lambda n, f: (n, 0)),
            pl.BlockSpec((_D, _TF), lambda n, f: (0, f)),
            pl.BlockSpec((1, _TF), lambda n, f: (0, f)),
            pl.BlockSpec((_TF, _E), lambda n, f: (f, 0)),
            pl.BlockSpec((1, _E), lambda n, f: (0, 0)),
            pl.BlockSpec((_MC, _TN, _TF), lambda n, f: (0, n, f)),
        ],
        out_specs=[
            pl.BlockSpec((_TN, _E), lambda n, f: (n, 0)),
            pl.BlockSpec((_TN, 1), lambda n, f: (n, 0)),
        ],
        out_shape=[
            jax.ShapeDtypeStruct((_N, _E), jnp.float32),
            jax.ShapeDtypeStruct((_N, 1), jnp.int32),
        ],
        scratch_shapes=[pltpu.VMEM((_MC, _TN, _E), jnp.float32)],
        compiler_params=pltpu.CompilerParams(
            dimension_semantics=("parallel", "arbitrary"),
        ),
    )(z, W1, b1r, W2, b2r, masks)
    return idx.reshape(_N), ent
